# baseline (speedup 1.0000x reference)
import jax
import jax.numpy as jnp
from jax.experimental import pallas as pl
from jax.experimental.pallas import tpu as pltpu

_NB = 4
_RB = 128


def _som_dist_kernel(x_hbm, w_hbm, o_hbm, x_v, w_v, o_v,
                     sem_x, sem_w, sem_o):
    cp_w = pltpu.make_async_copy(w_hbm, w_v, sem_w)
    cp_w.start()
    cp_x = []
    for r in range(_NB):
        cp = pltpu.make_async_copy(
            x_hbm.at[pl.ds(r * _RB, _RB), :],
            x_v.at[pl.ds(r * _RB, _RB), :],
            sem_x.at[r],
        )
        cp.start()
        cp_x.append(cp)

    cp_w.wait()
    w = w_v[...]
    w2 = jnp.sum(w * w, axis=1, keepdims=True).T

    cp_o = []
    for r in range(_NB):
        cp_x[r].wait()
        x = x_v[pl.ds(r * _RB, _RB), :]
        xm2 = x * -2.0
        x2 = jnp.sum(x * x, axis=1, keepdims=True)
        xw = jax.lax.dot_general(
            xm2, w,
            dimension_numbers=(((1,), (1,)), ((), ())),
            preferred_element_type=jnp.float32,
        )
        o_v[pl.ds(r * _RB, _RB), :] = (x2 + w2) + xw
        cp = pltpu.make_async_copy(
            o_v.at[pl.ds(r * _RB, _RB), :],
            o_hbm.at[pl.ds(r * _RB, _RB), :],
            sem_o.at[r],
        )
        cp.start()
        cp_o.append(cp)

    for r in range(_NB):
        cp_o[r].wait()


def kernel(x, weights):
    B, D = x.shape
    R, C, _ = weights.shape
    N = R * C
    w = weights.reshape(N, D)
    out = pl.pallas_call(
        _som_dist_kernel,
        in_specs=[
            pl.BlockSpec(memory_space=pl.ANY),
            pl.BlockSpec(memory_space=pl.ANY),
        ],
        out_specs=pl.BlockSpec(memory_space=pl.ANY),
        out_shape=jax.ShapeDtypeStruct((B, N), jnp.float32),
        scratch_shapes=[
            pltpu.VMEM((B, D), jnp.float32),
            pltpu.VMEM((N, D), jnp.float32),
            pltpu.VMEM((B, N), jnp.float32),
            pltpu.SemaphoreType.DMA((_NB,)),
            pltpu.SemaphoreType.DMA,
            pltpu.SemaphoreType.DMA((_NB,)),
        ],
    )(x, w)
    return out.reshape(B, R, C)
